# baseline (speedup 1.0000x reference)
import functools

import jax
import jax.numpy as jnp
from jax.experimental import pallas as pl
from jax.experimental.pallas import tpu as pltpu

M = 16384
D = 512
B = 4096


BB = 2048
BM = 512
NB = B // BB
NM = M // BM

_I32_MAX = jnp.iinfo(jnp.int32).max


def _lt(av, ai, bv, bi):
    return (av < bv) | ((av == bv) & (ai < bi))


def _top2_kernel(s_ref, v_ref, b_ref, sidx_ref, bd_ref, sd_ref,
                 s2_ref, m1_ref, i1_ref, m2_ref, i2_ref):
    mi = pl.program_id(1)
    samples = s_ref[...]

    @pl.when(mi == 0)
    def _init():
        ones = jnp.ones((1, D), jnp.float32)
        s2_ref[...] = jax.lax.dot_general(
            ones, samples * samples, (((1,), (1,)), ((), ())),
            preferred_element_type=jnp.float32)
        m1_ref[...] = jnp.full((1, BB), jnp.inf, jnp.float32)
        m2_ref[...] = jnp.full((1, BB), jnp.inf, jnp.float32)
        i1_ref[...] = jnp.zeros((1, BB), jnp.int32)
        i2_ref[...] = jnp.zeros((1, BB), jnp.int32)

    vblk = v_ref[...]
    onesd = jnp.ones((1, D), jnp.float32)
    v2 = jax.lax.dot_general(
        vblk * vblk, onesd, (((1,), (1,)), ((), ())),
        preferred_element_type=jnp.float32)
    sv = jax.lax.dot_general(
        vblk, samples, (((1,), (1,)), ((), ())),
        preferred_element_type=jnp.float32)
    d2 = (s2_ref[...] + v2) - 2.0 * sv
    d2 = jnp.maximum(d2, 0.0)

    gidx = (jax.lax.broadcasted_iota(jnp.int32, (BM, BB), 0) + mi * BM)
    n1 = jnp.min(d2, axis=0, keepdims=True)
    j1 = jnp.min(jnp.where(d2 == n1, gidx, _I32_MAX), axis=0, keepdims=True)
    dmask = jnp.where(gidx == j1, jnp.inf, d2)
    n2 = jnp.min(dmask, axis=0, keepdims=True)
    j2 = jnp.min(jnp.where(dmask == n2, gidx, _I32_MAX), axis=0, keepdims=True)

    m1, i1 = m1_ref[...], i1_ref[...]
    m2, i2 = m2_ref[...], i2_ref[...]
    first_old = _lt(m1, i1, n1, j1)
    f_v = jnp.where(first_old, m1, n1)
    f_i = jnp.where(first_old, i1, j1)
    lose_v = jnp.where(first_old, n1, m1)
    lose_i = jnp.where(first_old, j1, i1)
    sec_old = _lt(m2, i2, n2, j2)
    alt_v = jnp.where(sec_old, m2, n2)
    alt_i = jnp.where(sec_old, i2, j2)
    take_lose = _lt(lose_v, lose_i, alt_v, alt_i)
    s_v = jnp.where(take_lose, lose_v, alt_v)
    s_i = jnp.where(take_lose, lose_i, alt_i)
    m1_ref[...], i1_ref[...] = f_v, f_i
    m2_ref[...], i2_ref[...] = s_v, s_i

    @pl.when(mi == NM - 1)
    def _emit():
        b_ref[...] = i1_ref[...]
        sidx_ref[...] = i2_ref[...]
        bd_ref[...] = jnp.sqrt(m1_ref[...])
        sd_ref[...] = jnp.sqrt(m2_ref[...])


def _top2(samples, V):
    return pl.pallas_call(
        _top2_kernel,
        grid=(NB, NM),
        in_specs=[
            pl.BlockSpec((BB, D), lambda bi, mi: (bi, 0)),
            pl.BlockSpec((BM, D), lambda bi, mi: (mi, 0)),
        ],
        out_specs=[
            pl.BlockSpec((1, BB), lambda bi, mi: (0, bi)),
            pl.BlockSpec((1, BB), lambda bi, mi: (0, bi)),
            pl.BlockSpec((1, BB), lambda bi, mi: (0, bi)),
            pl.BlockSpec((1, BB), lambda bi, mi: (0, bi)),
        ],
        out_shape=[
            jax.ShapeDtypeStruct((1, B), jnp.int32),
            jax.ShapeDtypeStruct((1, B), jnp.int32),
            jax.ShapeDtypeStruct((1, B), jnp.float32),
            jax.ShapeDtypeStruct((1, B), jnp.float32),
        ],
        scratch_shapes=[
            pltpu.VMEM((1, BB), jnp.float32),
            pltpu.VMEM((1, BB), jnp.float32),
            pltpu.VMEM((1, BB), jnp.int32),
            pltpu.VMEM((1, BB), jnp.float32),
            pltpu.VMEM((1, BB), jnp.int32),
        ],
        compiler_params=pltpu.CompilerParams(
            dimension_semantics=("parallel", "arbitrary")),
    )(samples, V)


BR = 512
NR = M // BR


def _update_kernel(v_ref, sb_ref, bc_ref, sc_ref, sdc_ref, br_ref, eps_ref,
                   n_ref, t_ref, vout_ref, nout_ref, tout_ref):
    ri = pl.program_id(0)
    rows_row = (jax.lax.broadcasted_iota(jnp.int32, (1, BR), 1) + ri * BR)
    rows_col = (jax.lax.broadcasted_iota(jnp.int32, (BR, 1), 0) + ri * BR)
    b_col = bc_ref[...]
    s_col = sc_ref[...]
    b_row = br_ref[...]
    maskb = b_col == rows_row
    masks = s_col == rows_row
    onef = jnp.float32(1.0)
    cb = jnp.sum(jnp.where(maskb, onef, 0.0), axis=0, keepdims=True)
    cs = jnp.sum(jnp.where(masks, onef, 0.0), axis=0, keepdims=True)
    nout_ref[0] = n_ref[0] + cb + cs

    irow = jax.lax.broadcasted_iota(jnp.int32, (B, 1), 0)
    imax = jnp.max(jnp.where(maskb, irow, -1), axis=0, keepdims=True)
    tval = jnp.sum(jnp.where(maskb & (irow == imax), sdc_ref[...], 0.0),
                   axis=0, keepdims=True)
    tout_ref[0] = jnp.where(imax >= 0, tval, t_ref[0])

    onehot_t = (rows_col == b_row).astype(jnp.bfloat16)
    aug = jax.lax.dot_general(
        onehot_t, sb_ref[...], (((1,), (0,)), ((), ())),
        preferred_element_type=jnp.float32)
    ssum = aug[:, :D]
    cb_col = aug[:, D:D + 1]
    eps = eps_ref[0, 0]
    vout_ref[...] = v_ref[...] * (1.0 - eps * cb_col) + eps * ssum


def _update(V, samples_aug, b_col, s_col, sd_col, b_row, eps, n3, t3):
    return pl.pallas_call(
        _update_kernel,
        grid=(NR,),
        in_specs=[
            pl.BlockSpec((BR, D), lambda ri: (ri, 0)),
            pl.BlockSpec((B, D + 128), lambda ri: (0, 0)),
            pl.BlockSpec((B, 1), lambda ri: (0, 0)),
            pl.BlockSpec((B, 1), lambda ri: (0, 0)),
            pl.BlockSpec((B, 1), lambda ri: (0, 0)),
            pl.BlockSpec((1, B), lambda ri: (0, 0)),
            pl.BlockSpec((1, 1), lambda ri: (0, 0),
                         memory_space=pltpu.SMEM),
            pl.BlockSpec((1, 1, BR), lambda ri: (ri, 0, 0)),
            pl.BlockSpec((1, 1, BR), lambda ri: (ri, 0, 0)),
        ],
        out_specs=[
            pl.BlockSpec((BR, D), lambda ri: (ri, 0)),
            pl.BlockSpec((1, 1, BR), lambda ri: (ri, 0, 0)),
            pl.BlockSpec((1, 1, BR), lambda ri: (ri, 0, 0)),
        ],
        out_shape=[
            jax.ShapeDtypeStruct((M, D), jnp.float32),
            jax.ShapeDtypeStruct((NR, 1, BR), jnp.float32),
            jax.ShapeDtypeStruct((NR, 1, BR), jnp.float32),
        ],
        compiler_params=pltpu.CompilerParams(
            dimension_semantics=("arbitrary",)),
    )(V, samples_aug, b_col, s_col, sd_col, b_row, eps, n3, t3)


def kernel(it, samples, labels, V, n, t):
    del labels
    eps_b = jnp.asarray(1.0 / (it + 2), jnp.float32).reshape(1, 1)

    b_row, s_row, bd_row, sd_row = _top2(samples, V)

    b_col = b_row.reshape(B, 1)
    s_col = s_row.reshape(B, 1)
    sd_col = sd_row.reshape(B, 1)
    n3 = n.reshape(NR, 1, BR)
    t3 = t.reshape(NR, 1, BR)
    pad = jnp.zeros((B, 128), jnp.bfloat16).at[:, 0].set(jnp.bfloat16(1.0))
    samples_aug = jnp.concatenate([samples.astype(jnp.bfloat16), pad], axis=1)

    V_new, n_new3, t_new3 = _update(
        V, samples_aug, b_col, s_col, sd_col, b_row, eps_b, n3, t3)

    return (V_new, n_new3.reshape(M), t_new3.reshape(M),
            bd_row.reshape(B), sd_row.reshape(B))


# speedup vs baseline: 1.1501x; 1.1501x over previous
import dataclasses
import functools

import jax
import jax.numpy as jnp
from jax import lax
from jax.experimental import pallas as pl
from jax.experimental.pallas import tpu as pltpu
from jax.experimental.pallas import tpu_sc as plsc

M = 16384
D = 512
B = 4096


BB = 2048
BM = 512
NB = B // BB
NM = M // BM

_I32_MAX = jnp.iinfo(jnp.int32).max


def _lt(av, ai, bv, bi):
    return (av < bv) | ((av == bv) & (ai < bi))


def _top2_kernel(s_ref, v_ref, b_ref, sidx_ref, bd_ref, sd_ref,
                 s2_ref, m1_ref, i1_ref, m2_ref, i2_ref):
    mi = pl.program_id(1)
    samples = s_ref[...]

    @pl.when(mi == 0)
    def _init():
        ones = jnp.ones((1, D), jnp.float32)
        s2_ref[...] = jax.lax.dot_general(
            ones, samples * samples, (((1,), (1,)), ((), ())),
            preferred_element_type=jnp.float32)
        m1_ref[...] = jnp.full((1, BB), jnp.inf, jnp.float32)
        m2_ref[...] = jnp.full((1, BB), jnp.inf, jnp.float32)
        i1_ref[...] = jnp.zeros((1, BB), jnp.int32)
        i2_ref[...] = jnp.zeros((1, BB), jnp.int32)

    vblk = v_ref[...]
    onesd = jnp.ones((1, D), jnp.float32)
    v2 = jax.lax.dot_general(
        vblk * vblk, onesd, (((1,), (1,)), ((), ())),
        preferred_element_type=jnp.float32)
    sv = jax.lax.dot_general(
        vblk, samples, (((1,), (1,)), ((), ())),
        preferred_element_type=jnp.float32)
    d2 = (s2_ref[...] + v2) - 2.0 * sv
    d2 = jnp.maximum(d2, 0.0)

    gidx = (jax.lax.broadcasted_iota(jnp.int32, (BM, BB), 0) + mi * BM)
    n1 = jnp.min(d2, axis=0, keepdims=True)
    j1 = jnp.min(jnp.where(d2 == n1, gidx, _I32_MAX), axis=0, keepdims=True)
    dmask = jnp.where(gidx == j1, jnp.inf, d2)
    n2 = jnp.min(dmask, axis=0, keepdims=True)
    j2 = jnp.min(jnp.where(dmask == n2, gidx, _I32_MAX), axis=0, keepdims=True)

    m1, i1 = m1_ref[...], i1_ref[...]
    m2, i2 = m2_ref[...], i2_ref[...]
    first_old = _lt(m1, i1, n1, j1)
    f_v = jnp.where(first_old, m1, n1)
    f_i = jnp.where(first_old, i1, j1)
    lose_v = jnp.where(first_old, n1, m1)
    lose_i = jnp.where(first_old, j1, i1)
    sec_old = _lt(m2, i2, n2, j2)
    alt_v = jnp.where(sec_old, m2, n2)
    alt_i = jnp.where(sec_old, i2, j2)
    take_lose = _lt(lose_v, lose_i, alt_v, alt_i)
    s_v = jnp.where(take_lose, lose_v, alt_v)
    s_i = jnp.where(take_lose, lose_i, alt_i)
    m1_ref[...], i1_ref[...] = f_v, f_i
    m2_ref[...], i2_ref[...] = s_v, s_i

    @pl.when(mi == NM - 1)
    def _emit():
        b_ref[...] = i1_ref[...]
        sidx_ref[...] = i2_ref[...]
        bd_ref[...] = jnp.sqrt(m1_ref[...])
        sd_ref[...] = jnp.sqrt(m2_ref[...])


def _top2(samples, V):
    return pl.pallas_call(
        _top2_kernel,
        grid=(NB, NM),
        in_specs=[
            pl.BlockSpec((BB, D), lambda bi, mi: (bi, 0)),
            pl.BlockSpec((BM, D), lambda bi, mi: (mi, 0)),
        ],
        out_specs=[
            pl.BlockSpec((1, BB), lambda bi, mi: (0, bi)),
            pl.BlockSpec((1, BB), lambda bi, mi: (0, bi)),
            pl.BlockSpec((1, BB), lambda bi, mi: (0, bi)),
            pl.BlockSpec((1, BB), lambda bi, mi: (0, bi)),
        ],
        out_shape=[
            jax.ShapeDtypeStruct((1, B), jnp.int32),
            jax.ShapeDtypeStruct((1, B), jnp.int32),
            jax.ShapeDtypeStruct((1, B), jnp.float32),
            jax.ShapeDtypeStruct((1, B), jnp.float32),
        ],
        scratch_shapes=[
            pltpu.VMEM((1, BB), jnp.float32),
            pltpu.VMEM((1, BB), jnp.float32),
            pltpu.VMEM((1, BB), jnp.int32),
            pltpu.VMEM((1, BB), jnp.float32),
            pltpu.VMEM((1, BB), jnp.int32),
        ],
        compiler_params=pltpu.CompilerParams(
            dimension_semantics=("parallel", "arbitrary")),
    )(samples, V)


BI = 512
NI = B // BI


def _prep_kernel(bc_ref, sc_ref, br_ref, sr_ref,
                 cntb_ref, cnts_ref, winb_ref, wins_ref):
    ii = pl.program_id(0)
    b_col = bc_ref[...]
    s_col = sc_ref[...]
    b_row = br_ref[...]
    s_row = sr_ref[...]
    irow = jax.lax.broadcasted_iota(jnp.int32, (B, 1), 0)
    icol = jax.lax.broadcasted_iota(jnp.int32, (1, BI), 1) + ii * BI
    onef = jnp.float32(1.0)

    eb = b_col == b_row
    cntb_ref[...] = jnp.sum(jnp.where(eb, onef, 0.0), axis=0, keepdims=True)
    later_b = jnp.sum(jnp.where(eb & (irow > icol), onef, 0.0),
                      axis=0, keepdims=True)
    winb_ref[...] = (later_b == 0.0).astype(jnp.int32)

    es = s_col == s_row
    cnts_ref[...] = jnp.sum(jnp.where(es, onef, 0.0), axis=0, keepdims=True)
    later_s = jnp.sum(jnp.where(es & (irow > icol), onef, 0.0),
                      axis=0, keepdims=True)
    wins_ref[...] = (later_s == 0.0).astype(jnp.int32)


def _prep(b_col, s_col, b_row, s_row):
    return pl.pallas_call(
        _prep_kernel,
        grid=(NI,),
        in_specs=[
            pl.BlockSpec((B, 1), lambda ii: (0, 0)),
            pl.BlockSpec((B, 1), lambda ii: (0, 0)),
            pl.BlockSpec((1, BI), lambda ii: (0, ii)),
            pl.BlockSpec((1, BI), lambda ii: (0, ii)),
        ],
        out_specs=[
            pl.BlockSpec((1, BI), lambda ii: (0, ii)),
            pl.BlockSpec((1, BI), lambda ii: (0, ii)),
            pl.BlockSpec((1, BI), lambda ii: (0, ii)),
            pl.BlockSpec((1, BI), lambda ii: (0, ii)),
        ],
        out_shape=[
            jax.ShapeDtypeStruct((1, B), jnp.float32),
            jax.ShapeDtypeStruct((1, B), jnp.float32),
            jax.ShapeDtypeStruct((1, B), jnp.int32),
            jax.ShapeDtypeStruct((1, B), jnp.int32),
        ],
        compiler_params=pltpu.CompilerParams(
            dimension_semantics=("arbitrary",)),
    )(b_col, s_col, b_row, s_row)


_CH = 16


def _nt_kernel(b_hbm, s_hbm, sd_hbm, winb_hbm, wins_hbm, cntb_hbm, cnts_hbm,
               n_hbm, t_hbm, nout_hbm, tout_hbm,
               acc_v, b_v, s_v, val_v, win_v, win2_v):
    cid = lax.axis_index("c")
    sid = lax.axis_index("s")
    wid = sid * 2 + cid

    @pl.when(wid == 0)
    def _do_n():
        pltpu.sync_copy(n_hbm, acc_v)
        pltpu.sync_copy(b_hbm, b_v)
        pltpu.sync_copy(s_hbm, s_v)
        pltpu.sync_copy(cntb_hbm, val_v)
        pltpu.sync_copy(winb_hbm, win_v)
        pltpu.sync_copy(wins_hbm, win2_v)

        @pl.loop(0, B // _CH)
        def _bphase(k):
            sl = pl.ds(k * _CH, _CH)
            idx = b_v[sl]
            val = val_v[sl]
            mask = win_v[sl] != 0
            plsc.addupdate_scatter(acc_v, [idx], val, mask=mask)

        pltpu.sync_copy(cnts_hbm, val_v)

        @pl.loop(0, B // _CH)
        def _sphase(k):
            sl = pl.ds(k * _CH, _CH)
            idx = s_v[sl]
            val = val_v[sl]
            mask = win2_v[sl] != 0
            plsc.addupdate_scatter(acc_v, [idx], val, mask=mask)

        pltpu.sync_copy(acc_v, nout_hbm)

    @pl.when(wid == 1)
    def _do_t():
        pltpu.sync_copy(t_hbm, acc_v)
        pltpu.sync_copy(b_hbm, b_v)
        pltpu.sync_copy(sd_hbm, val_v)
        pltpu.sync_copy(winb_hbm, win_v)

        @pl.loop(0, B // _CH)
        def _tphase(k):
            sl = pl.ds(k * _CH, _CH)
            idx = b_v[sl]
            val = val_v[sl]
            mask = win_v[sl] != 0
            plsc.store_scatter(acc_v, [idx], val, mask=mask)

        pltpu.sync_copy(acc_v, tout_hbm)


def _nt_scatter(b_flat, s_flat, sd_flat, winb, wins, cntb, cnts, n, t):
    mesh = plsc.VectorSubcoreMesh(core_axis_name="c", subcore_axis_name="s")
    cp = pltpu.CompilerParams()
    if "needs_layout_passes" in pltpu.CompilerParams.__dataclass_fields__:
        cp = dataclasses.replace(cp, needs_layout_passes=False)
    f = pl.kernel(
        _nt_kernel,
        out_type=[
            jax.ShapeDtypeStruct((M,), jnp.float32),
            jax.ShapeDtypeStruct((M,), jnp.float32),
        ],
        mesh=mesh,
        scratch_types=[
            pltpu.VMEM((M,), jnp.float32),
            pltpu.VMEM((B,), jnp.int32),
            pltpu.VMEM((B,), jnp.int32),
            pltpu.VMEM((B,), jnp.float32),
            pltpu.VMEM((B,), jnp.int32),
            pltpu.VMEM((B,), jnp.int32),
        ],
        compiler_params=cp,
    )
    return f(b_flat, s_flat, sd_flat, winb, wins, cntb, cnts, n, t)


BR = 512
NR = M // BR


def _v_kernel(v_ref, sb_ref, br_ref, eps_ref, vout_ref):
    ri = pl.program_id(0)
    rows_col = (jax.lax.broadcasted_iota(jnp.int32, (BR, 1), 0) + ri * BR)
    b_row = br_ref[...]
    onehot_t = (rows_col == b_row).astype(jnp.bfloat16)
    aug = jax.lax.dot_general(
        onehot_t, sb_ref[...], (((1,), (0,)), ((), ())),
        preferred_element_type=jnp.float32)
    ssum = aug[:, :D]
    cb_col = aug[:, D:D + 1]
    eps = eps_ref[0, 0]
    vout_ref[...] = v_ref[...] * (1.0 - eps * cb_col) + eps * ssum


def _v_update(V, samples_aug, b_row, eps):
    return pl.pallas_call(
        _v_kernel,
        grid=(NR,),
        in_specs=[
            pl.BlockSpec((BR, D), lambda ri: (ri, 0)),
            pl.BlockSpec((B, D + 128), lambda ri: (0, 0)),
            pl.BlockSpec((1, B), lambda ri: (0, 0)),
            pl.BlockSpec((1, 1), lambda ri: (0, 0),
                         memory_space=pltpu.SMEM),
        ],
        out_specs=pl.BlockSpec((BR, D), lambda ri: (ri, 0)),
        out_shape=jax.ShapeDtypeStruct((M, D), jnp.float32),
        compiler_params=pltpu.CompilerParams(
            dimension_semantics=("arbitrary",)),
    )(V, samples_aug, b_row, eps)


def kernel(it, samples, labels, V, n, t):
    del labels
    eps_b = jnp.asarray(1.0 / (it + 2), jnp.float32).reshape(1, 1)

    b_row, s_row, bd_row, sd_row = _top2(samples, V)

    b_col = b_row.reshape(B, 1)
    s_col = s_row.reshape(B, 1)
    cntb, cnts, winb, wins = _prep(b_col, s_col, b_row, s_row)

    n_new, t_new = _nt_scatter(
        b_row.reshape(B), s_row.reshape(B), sd_row.reshape(B),
        winb.reshape(B), wins.reshape(B), cntb.reshape(B), cnts.reshape(B),
        n, t)

    pad = jnp.zeros((B, 128), jnp.bfloat16).at[:, 0].set(jnp.bfloat16(1.0))
    samples_aug = jnp.concatenate([samples.astype(jnp.bfloat16), pad], axis=1)
    V_new = _v_update(V, samples_aug, b_row, eps_b)

    return (V_new, n_new, t_new, bd_row.reshape(B), sd_row.reshape(B))
